# baseline (speedup 1.0000x reference)
import functools
import math

import jax
import jax.numpy as jnp
from jax import lax
from jax.experimental import pallas as pl
from jax.experimental.pallas import tpu as pltpu
from jax.experimental.pallas import tpu_sc as plsc

_K = 512
_L = 64
_B = 4096
_LANES = 128
_RB = 64

_B_SC = 1024
_B_TC = _B - _B_SC
_T_ROWS = _B_TC * _L // _LANES
_B_W = _B_SC // 32
_E_W = _B_W * _L

_HALF_LOG_2PI = 0.5 * math.log(2.0 * math.pi)
_LOG2E = math.log2(math.e)
_LN2 = math.log(2.0)


def _coef_sc_kernel(m_ref, lv_ref, w_ref, an_ref, bn_ref, cn_ref, mn_ref):
    lv = lv_ref[...]
    mu = m_ref[...]
    wv = w_ref[...]
    wmax = jnp.max(wv, axis=0, keepdims=True)
    lse_w = wmax + jnp.log(jnp.sum(jnp.exp(wv - wmax), axis=0, keepdims=True))
    lw = wv - lse_w
    a2 = -0.5 * jnp.exp(-lv)
    c0 = (lw - _HALF_LOG_2PI) - 0.5 * lv
    cap = jnp.max(c0, axis=0, keepdims=True)
    an_ref[...] = a2
    bn_ref[...] = -2.0 * a2 * mu
    cn_ref[...] = (c0 - cap) + a2 * mu * mu
    mn_ref[...] = cap


def _coef_tc_kernel(mt_ref, lvt_ref, w_ref, a2_ref, b2_ref, c2_ref, m2_ref):
    lv = lvt_ref[...]
    mu = mt_ref[...]
    wv = w_ref[...]
    wmax = jnp.max(wv, axis=0, keepdims=True)
    lse_w = wmax + jnp.log(jnp.sum(jnp.exp(wv - wmax), axis=0, keepdims=True))
    lw = wv - lse_w
    a2 = -0.5 * jnp.exp(-lv)
    c0 = _LOG2E * ((lw - _HALF_LOG_2PI) - 0.5 * lv)
    cap = jnp.max(c0, axis=0, keepdims=True)
    av = _LOG2E * a2
    bv = _LOG2E * (-2.0 * a2) * mu
    cv = (c0 - cap) + (_LOG2E * a2) * mu * mu
    a2_ref[...] = jnp.broadcast_to(av[:, None, :], (_K, 8, _LANES))
    b2_ref[...] = jnp.broadcast_to(bv[:, None, :], (_K, 8, _LANES))
    c2_ref[...] = jnp.broadcast_to(cv[:, None, :], (_K, 8, _LANES))
    m2_ref[...] = cap


def _mog_kernel(z_ref, a_ref, b_ref, c_ref, m_ref, out_ref):
    cap = m_ref[...]
    for g in range(0, _RB, 8):
        z8 = z_ref[g:g + 8, :]
        accs = [None] * 4
        for k in range(_K):
            t = jnp.exp2(c_ref[k] + z8 * (b_ref[k] + z8 * a_ref[k]))
            i = k % 4
            accs[i] = t if accs[i] is None else accs[i] + t
        s = (accs[0] + accs[1]) + (accs[2] + accs[3])
        s = jnp.maximum(s, 2.0 ** -140)
        out_ref[g:g + 8, :] = _LN2 * (cap + jnp.log2(s))


def _sc_body(z_hbm, a_hbm, b_hbm, c_hbm, s_hbm, z_v, a_v, b_v, c_v, s_v):
    cid = lax.axis_index("c")
    sid = lax.axis_index("s")
    wid = sid * 2 + cid
    pltpu.sync_copy(a_hbm, a_v)
    pltpu.sync_copy(b_hbm, b_v)
    pltpu.sync_copy(c_hbm, c_v)
    base = wid * _E_W
    pltpu.sync_copy(z_hbm.at[pl.ds(base, _E_W)], z_v)
    for bb in range(_B_W // 8):
        for q in range(4):
            zs = [z_v[pl.ds((bb * 8 + t) * _L + q * 16, 16)] for t in range(8)]

            def body(k, accs, _q=q):
                ak = a_v[pl.ds(k * _L + _q * 16, 16)]
                bk = b_v[pl.ds(k * _L + _q * 16, 16)]
                ck = c_v[pl.ds(k * _L + _q * 16, 16)]
                return tuple(acc + jnp.exp(ck + zt * (bk + zt * ak))
                             for acc, zt in zip(accs, zs))

            accs0 = tuple(jnp.zeros((16,), jnp.float32) for _ in range(8))
            accs = lax.fori_loop(0, _K, body, accs0)
            for t in range(8):
                s_v[pl.ds((bb * 8 + t) * _L + q * 16, 16)] = accs[t]
    pltpu.sync_copy(s_v, s_hbm.at[pl.ds(base, _E_W)])


def _epi_kernel(s_ref, m_ref, out_ref):
    s = jnp.maximum(s_ref[...], 1e-38)
    out_ref[...] = m_ref[...] + jnp.log(s)


def kernel(z, means, logvars, w):
    zf = z.reshape(_B * _L)
    z2 = z.reshape(_B * _L // _LANES, _LANES)
    mt = jnp.concatenate([means, means], axis=1)
    lvt = jnp.concatenate([logvars, logvars], axis=1)
    wc = w.reshape(_K, 1)
    kl = jax.ShapeDtypeStruct((_K, _LANES), jnp.float32)
    onel = jax.ShapeDtypeStruct((1, _LANES), jnp.float32)
    kh = jax.ShapeDtypeStruct((_K, _L), jnp.float32)
    oneh = jax.ShapeDtypeStruct((1, _L), jnp.float32)

    an, bn, cn, mn = pl.pallas_call(
        _coef_sc_kernel,
        out_shape=(kh, kh, kh, oneh),
    )(means, logvars, wc)
    sc_run = functools.partial(
        pl.kernel,
        out_type=jax.ShapeDtypeStruct((_B_SC * _L,), jnp.float32),
        mesh=plsc.VectorSubcoreMesh(core_axis_name="c", subcore_axis_name="s"),
        scratch_types=[
            pltpu.VMEM((_E_W,), jnp.float32),
            pltpu.VMEM((_K * _L,), jnp.float32),
            pltpu.VMEM((_K * _L,), jnp.float32),
            pltpu.VMEM((_K * _L,), jnp.float32),
            pltpu.VMEM((_E_W,), jnp.float32),
        ],
    )(_sc_body)
    s_flat = sc_run(zf[_B_TC * _L:], an.reshape(-1), bn.reshape(-1),
                    cn.reshape(-1))
    cap128 = jnp.concatenate([mn, mn], axis=1)
    out_sc = pl.pallas_call(
        _epi_kernel,
        out_shape=jax.ShapeDtypeStruct((_B_SC * _L // _LANES, _LANES), jnp.float32),
    )(s_flat.reshape(_B_SC * _L // _LANES, _LANES), cap128)

    kle = jax.ShapeDtypeStruct((_K, 8, _LANES), jnp.float32)
    a2, b2, c2, m2 = pl.pallas_call(
        _coef_tc_kernel,
        out_shape=(kle, kle, kle, onel),
    )(mt, lvt, wc)
    out_tc = pl.pallas_call(
        _mog_kernel,
        grid=(_T_ROWS // _RB,),
        in_specs=[
            pl.BlockSpec((_RB, _LANES), lambda i: (i, 0)),
            pl.BlockSpec((_K, 8, _LANES), lambda i: (0, 0, 0)),
            pl.BlockSpec((_K, 8, _LANES), lambda i: (0, 0, 0)),
            pl.BlockSpec((_K, 8, _LANES), lambda i: (0, 0, 0)),
            pl.BlockSpec((1, _LANES), lambda i: (0, 0)),
        ],
        out_specs=pl.BlockSpec((_RB, _LANES), lambda i: (i, 0)),
        out_shape=jax.ShapeDtypeStruct((_T_ROWS, _LANES), jnp.float32),
    )(z2[:_T_ROWS, :], a2, b2, c2, m2)

    out2 = jnp.concatenate([out_tc, out_sc], axis=0)
    return out2.reshape(_B, _L)


# speedup vs baseline: 1.1812x; 1.1812x over previous
import functools
import math

import jax
import jax.numpy as jnp
from jax import lax
from jax.experimental import pallas as pl
from jax.experimental.pallas import tpu as pltpu
from jax.experimental.pallas import tpu_sc as plsc

_K = 512
_L = 64
_B = 4096
_LANES = 128
_RB = 64

_B_SC = 0
_B_TC = _B - _B_SC
_T_ROWS = _B_TC * _L // _LANES
_SC_ROWS = _B_SC * _L // _LANES
_E_W = _SC_ROWS * _LANES // 32

_HALF_LOG_2PI = 0.5 * math.log(2.0 * math.pi)
_LOG2E = math.log2(math.e)
_LN2 = math.log(2.0)


def _coef_sc_kernel(m_ref, lv_ref, w_ref, an_ref, bn_ref, cn_ref, mn_ref):
    lv = lv_ref[...]
    mu = m_ref[...]
    wv = w_ref[...]
    wmax = jnp.max(wv, axis=0, keepdims=True)
    lse_w = wmax + jnp.log(jnp.sum(jnp.exp(wv - wmax), axis=0, keepdims=True))
    lw = wv - lse_w
    a2 = -0.5 * jnp.exp(-lv)
    c0 = (lw - _HALF_LOG_2PI) - 0.5 * lv
    cap = jnp.max(c0, axis=0, keepdims=True)
    an_ref[...] = a2
    bn_ref[...] = -2.0 * a2 * mu
    cn_ref[...] = (c0 - cap) + a2 * mu * mu
    mn_ref[...] = cap


def _coef_tc_kernel(mt_ref, lvt_ref, w_ref, a2_ref, b2_ref, c2_ref, m2_ref):
    lv = lvt_ref[...]
    mu = mt_ref[...]
    wv = w_ref[...]
    wmax = jnp.max(wv, axis=0, keepdims=True)
    lse_w = wmax + jnp.log(jnp.sum(jnp.exp(wv - wmax), axis=0, keepdims=True))
    lw = wv - lse_w
    a2 = -0.5 * jnp.exp(-lv)
    c0 = _LOG2E * ((lw - _HALF_LOG_2PI) - 0.5 * lv)
    cap = jnp.max(c0, axis=0, keepdims=True)
    av = _LOG2E * a2
    bv = _LOG2E * (-2.0 * a2) * mu
    cv = (c0 - cap) + (_LOG2E * a2) * mu * mu
    a2_ref[...] = jnp.broadcast_to(av[:, None, :], (_K, 8, _LANES))
    b2_ref[...] = jnp.broadcast_to(bv[:, None, :], (_K, 8, _LANES))
    c2_ref[...] = jnp.broadcast_to(cv[:, None, :], (_K, 8, _LANES))
    m2_ref[...] = cap


def _mog_kernel(z_ref, a_ref, b_ref, c_ref, m_ref, out_ref):
    cap = m_ref[...]
    for g in range(0, _RB, 8):
        z8 = z_ref[g:g + 8, :]
        accs = [None] * 4
        for k in range(_K):
            t = jnp.exp2(c_ref[k] + z8 * (b_ref[k] + z8 * a_ref[k]))
            i = k % 4
            accs[i] = t if accs[i] is None else accs[i] + t
        s = (accs[0] + accs[1]) + (accs[2] + accs[3])
        s = jnp.maximum(s, 2.0 ** -140)
        out_ref[g:g + 8, :] = _LN2 * (cap + jnp.log2(s))


def _sc_body(z_hbm, a_hbm, b_hbm, c_hbm, s_hbm, z_v, a_v, b_v, c_v, s_v):
    cid = lax.axis_index("c")
    sid = lax.axis_index("s")
    wid = sid * 2 + cid
    pltpu.sync_copy(a_hbm, a_v)
    pltpu.sync_copy(b_hbm, b_v)
    pltpu.sync_copy(c_hbm, c_v)
    base = wid * _E_W
    pltpu.sync_copy(z_hbm.at[pl.ds(base, _E_W)], z_v)
    for bb in range(_E_W // (8 * _L)):
        for q in range(4):
            zs = [z_v[pl.ds((bb * 8 + t) * _L + q * 16, 16)] for t in range(8)]

            def body(k, accs, _q=q):
                ak = a_v[pl.ds(k * _L + _q * 16, 16)]
                bk = b_v[pl.ds(k * _L + _q * 16, 16)]
                ck = c_v[pl.ds(k * _L + _q * 16, 16)]
                return tuple(acc + jnp.exp(ck + zt * (bk + zt * ak))
                             for acc, zt in zip(accs, zs))

            accs0 = tuple(jnp.zeros((16,), jnp.float32) for _ in range(8))
            accs = lax.fori_loop(0, _K, body, accs0)
            for t in range(8):
                s_v[pl.ds((bb * 8 + t) * _L + q * 16, 16)] = accs[t]
    pltpu.sync_copy(s_v, s_hbm.at[pl.ds(base, _E_W)])


def _epi_kernel(s_ref, m_ref, out_ref):
    s = jnp.maximum(s_ref[...], 1e-38)
    out_ref[...] = m_ref[...] + jnp.log(s)


def kernel(z, means, logvars, w):
    z2 = z.reshape(_B * _L // _LANES, _LANES)
    mt = jnp.concatenate([means, means], axis=1)
    lvt = jnp.concatenate([logvars, logvars], axis=1)
    wc = w.reshape(_K, 1)
    kl = jax.ShapeDtypeStruct((_K, _LANES), jnp.float32)
    onel = jax.ShapeDtypeStruct((1, _LANES), jnp.float32)
    kh = jax.ShapeDtypeStruct((_K, _L), jnp.float32)
    oneh = jax.ShapeDtypeStruct((1, _L), jnp.float32)

    if _B_SC:
        an, bn, cn, mn = pl.pallas_call(
            _coef_sc_kernel,
            out_shape=(kh, kh, kh, oneh),
        )(means, logvars, wc)
        sc_run = functools.partial(
            pl.kernel,
            out_type=jax.ShapeDtypeStruct((_SC_ROWS * _LANES,), jnp.float32),
            mesh=plsc.VectorSubcoreMesh(core_axis_name="c", subcore_axis_name="s"),
            scratch_types=[
                pltpu.VMEM((_E_W,), jnp.float32),
                pltpu.VMEM((_K * _L,), jnp.float32),
                pltpu.VMEM((_K * _L,), jnp.float32),
                pltpu.VMEM((_K * _L,), jnp.float32),
                pltpu.VMEM((_E_W,), jnp.float32),
            ],
        )(_sc_body)
        s_flat = sc_run(z.reshape(_B * _L)[_B_TC * _L:], an.reshape(-1),
                        bn.reshape(-1), cn.reshape(-1))
        cap128 = jnp.concatenate([mn, mn], axis=1)
        out_sc = pl.pallas_call(
            _epi_kernel,
            out_shape=jax.ShapeDtypeStruct((_SC_ROWS, _LANES), jnp.float32),
        )(s_flat.reshape(_SC_ROWS, _LANES), cap128)

    kle = jax.ShapeDtypeStruct((_K, 8, _LANES), jnp.float32)
    a2, b2, c2, m2 = pl.pallas_call(
        _coef_tc_kernel,
        out_shape=(kle, kle, kle, onel),
    )(mt, lvt, wc)
    out_tc = pl.pallas_call(
        _mog_kernel,
        grid=(_T_ROWS // _RB,),
        in_specs=[
            pl.BlockSpec((_RB, _LANES), lambda i: (i, 0)),
            pl.BlockSpec((_K, 8, _LANES), lambda i: (0, 0, 0)),
            pl.BlockSpec((_K, 8, _LANES), lambda i: (0, 0, 0)),
            pl.BlockSpec((_K, 8, _LANES), lambda i: (0, 0, 0)),
            pl.BlockSpec((1, _LANES), lambda i: (0, 0)),
        ],
        out_specs=pl.BlockSpec((_RB, _LANES), lambda i: (i, 0)),
        out_shape=jax.ShapeDtypeStruct((_T_ROWS, _LANES), jnp.float32),
    )(z2[:_T_ROWS, :], a2, b2, c2, m2)

    out2 = jnp.concatenate([out_tc, out_sc], axis=0) if _B_SC else out_tc
    return out2.reshape(_B, _L)
